# baseline (speedup 1.0000x reference)
import functools

import jax
import jax.numpy as jnp
from jax import lax
from jax.experimental import pallas as pl
from jax.experimental.pallas import tpu as pltpu
from jax.experimental.pallas import tpu_sc as plsc

NUM_CORES = 2
NUM_SUBCORES = 16
NW = NUM_CORES * NUM_SUBCORES
LANES = 16


def _sc_encode(rid, table, posT, *, B, T, N, D):
    b_per_w = B // NW
    n_dc = D // LANES

    mesh = plsc.VectorSubcoreMesh(
        core_axis_name="c", subcore_axis_name="s",
        num_cores=NUM_CORES, num_subcores=NUM_SUBCORES)

    @functools.partial(
        pl.kernel,
        out_type=jax.ShapeDtypeStruct((B, D * T), jnp.float32),
        mesh=mesh,
        compiler_params=pltpu.CompilerParams(
            needs_layout_passes=False, use_tc_tiling_on_sc=False),
        scratch_types=[
            pltpu.VMEM((T,), jnp.int32),
            pltpu.VMEM((T, D), jnp.float32),
            pltpu.VMEM((T, D), jnp.float32),
            pltpu.VMEM((D * T,), jnp.float32),
            pltpu.SemaphoreType.DMA,
        ],
    )
    def sc_kernel(rid_hbm, table_hbm, posT_hbm, out_hbm,
                  idx_v, rows_v, posT_v, outT_v, sem):
        wid = lax.axis_index("s") * NUM_CORES + lax.axis_index("c")
        base = wid * b_per_w
        pltpu.sync_copy(posT_hbm, posT_v)
        iota = lax.iota(jnp.int32, LANES)
        dbases = [(iota + d0 * LANES) * T for d0 in range(n_dc)]

        def body(i, carry):
            b = base + i
            pltpu.sync_copy(rid_hbm.at[b], idx_v)
            cps = []
            off = 0
            while off < T:
                n = min(128, T - off)
                cps.append(pltpu.async_copy(
                    table_hbm.at[idx_v.at[pl.ds(off, n)]],
                    rows_v.at[pl.ds(off, n)], sem))
                off += n
            for cp in cps:
                cp.wait()

            def tbody(t, c2):
                for d0 in range(n_dc):
                    vec = (rows_v[t, pl.ds(d0 * LANES, LANES)]
                           + posT_v[t, pl.ds(d0 * LANES, LANES)])
                    plsc.store_scatter(outT_v, [dbases[d0] + t], vec)
                return c2

            lax.fori_loop(0, T, tbody, 0)
            pltpu.sync_copy(outT_v, out_hbm.at[b])
            return carry

        lax.fori_loop(0, b_per_w, body, 0)

    return sc_kernel(rid, table, posT)


def kernel(rid, table, pos):
    B, T = rid.shape
    N, D = table.shape
    rid = rid.astype(jnp.int32)
    posT = jnp.transpose(pos[0].astype(jnp.float32))
    out = _sc_encode(rid, table, posT, B=B, T=T, N=N, D=D)
    return out.reshape(B, D, T)


# speedup vs baseline: 1.1884x; 1.1884x over previous
import functools

import jax
import jax.numpy as jnp
from jax import lax
from jax.experimental import pallas as pl
from jax.experimental.pallas import tpu as pltpu
from jax.experimental.pallas import tpu_sc as plsc

NUM_CORES = 2
NUM_SUBCORES = 16
NW = NUM_CORES * NUM_SUBCORES
LANES = 16
GCHUNK = 128


def _sc_encode(rid, table, posT, *, B, T, N, D):
    b_per_w = B // NW
    n_dc = D // LANES
    chunks = []
    off = 0
    while off < T:
        chunks.append((off, min(GCHUNK, T - off)))
        off += chunks[-1][1]

    mesh = plsc.VectorSubcoreMesh(
        core_axis_name="c", subcore_axis_name="s",
        num_cores=NUM_CORES, num_subcores=NUM_SUBCORES)

    @functools.partial(
        pl.kernel,
        out_type=jax.ShapeDtypeStruct((B, D * T), jnp.float32),
        mesh=mesh,
        compiler_params=pltpu.CompilerParams(
            needs_layout_passes=False, use_tc_tiling_on_sc=False),
        scratch_types=[
            pltpu.VMEM((T,), jnp.int32),
            pltpu.VMEM((T,), jnp.int32),
            pltpu.VMEM((T, D), jnp.float32),
            pltpu.VMEM((T, D), jnp.float32),
            pltpu.VMEM((T, D), jnp.float32),
            pltpu.VMEM((D * T,), jnp.float32),
            pltpu.VMEM((D * T,), jnp.float32),
            pltpu.SemaphoreType.DMA,
            pltpu.SemaphoreType.DMA,
            pltpu.SemaphoreType.DMA,
            pltpu.SemaphoreType.DMA,
            pltpu.SemaphoreType.DMA,
            pltpu.SemaphoreType.DMA,
        ],
    )
    def sc_kernel(rid_hbm, table_hbm, posT_hbm, out_hbm,
                  idx0, idx1, rows0, rows1, posT_v, outT0, outT1,
                  semi0, semi1, semg0, semg1, semo0, semo1):
        idx_v = (idx0, idx1)
        rows_v = (rows0, rows1)
        outT_v = (outT0, outT1)
        semi = (semi0, semi1)
        semg = (semg0, semg1)
        semo = (semo0, semo1)

        wid = lax.axis_index("s") * NUM_CORES + lax.axis_index("c")
        base = wid * b_per_w
        pltpu.sync_copy(posT_hbm, posT_v)
        iota = lax.iota(jnp.int32, LANES)
        dbases = [(iota + d0 * LANES) * T for d0 in range(n_dc)]

        def start_gather(buf):
            for off, n in chunks:
                pltpu.async_copy(
                    table_hbm.at[idx_v[buf].at[pl.ds(off, n)]],
                    rows_v[buf].at[pl.ds(off, n)], semg[buf])

        def drain_gather(buf):
            pltpu.make_async_copy(
                table_hbm.at[pl.ds(0, T)], rows_v[buf], semg[buf]).wait()

        for buf in (0, 1):
            pltpu.sync_copy(rid_hbm.at[base + buf], idx_v[buf])
            start_gather(buf)

        def body(j, carry):
            for buf in (0, 1):
                b = base + 2 * j + buf
                bn = jnp.minimum(b + 2, B - 1)
                drain_gather(buf)
                pltpu.async_copy(rid_hbm.at[bn], idx_v[buf], semi[buf])
                @pl.when(j > 0)
                def _():
                    pltpu.make_async_copy(
                        outT_v[buf], out_hbm.at[b], semo[buf]).wait()

                def tbody(t, c2):
                    for d0 in range(n_dc):
                        vec = (rows_v[buf][t, pl.ds(d0 * LANES, LANES)]
                               + posT_v[t, pl.ds(d0 * LANES, LANES)])
                        plsc.store_scatter(outT_v[buf], [dbases[d0] + t], vec)
                    return c2

                lax.fori_loop(0, T, tbody, 0)
                pltpu.async_copy(outT_v[buf], out_hbm.at[b], semo[buf])
                pltpu.make_async_copy(
                    rid_hbm.at[bn], idx_v[buf], semi[buf]).wait()
                start_gather(buf)
            return carry

        lax.fori_loop(0, b_per_w // 2, body, 0)

        for buf in (0, 1):
            drain_gather(buf)
            pltpu.make_async_copy(
                outT_v[buf], out_hbm.at[base], semo[buf]).wait()

    return sc_kernel(rid, table, posT)


def kernel(rid, table, pos):
    B, T = rid.shape
    N, D = table.shape
    rid = rid.astype(jnp.int32)
    posT = jnp.transpose(pos[0].astype(jnp.float32))
    out = _sc_encode(rid, table, posT, B=B, T=T, N=N, D=D)
    return out.reshape(B, D, T)


# speedup vs baseline: 1.5771x; 1.3271x over previous
import functools

import jax
import jax.numpy as jnp
from jax import lax
from jax.experimental import pallas as pl
from jax.experimental.pallas import tpu as pltpu
from jax.experimental.pallas import tpu_sc as plsc

NUM_CORES = 2
NUM_SUBCORES = 16
NW = NUM_CORES * NUM_SUBCORES
LANES = 16
GCHUNK = 128


def _sc_encode(rid, table, posT, *, B, T, N, D):
    b_per_w = B // NW
    n_dc = D // LANES
    chunks = []
    off = 0
    while off < T:
        chunks.append((off, min(GCHUNK, T - off)))
        off += chunks[-1][1]

    mesh = plsc.VectorSubcoreMesh(
        core_axis_name="c", subcore_axis_name="s",
        num_cores=NUM_CORES, num_subcores=NUM_SUBCORES)

    @functools.partial(
        pl.kernel,
        out_type=jax.ShapeDtypeStruct((B, D * T), jnp.float32),
        mesh=mesh,
        compiler_params=pltpu.CompilerParams(
            needs_layout_passes=False, use_tc_tiling_on_sc=False),
        scratch_types=[
            pltpu.VMEM((T,), jnp.int32),
            pltpu.VMEM((T,), jnp.int32),
            pltpu.VMEM((T, D), jnp.float32),
            pltpu.VMEM((T, D), jnp.float32),
            pltpu.VMEM((T, D), jnp.float32),
            pltpu.VMEM((D * T,), jnp.float32),
            pltpu.VMEM((D * T,), jnp.float32),
            pltpu.SemaphoreType.DMA,
            pltpu.SemaphoreType.DMA,
            pltpu.SemaphoreType.DMA,
            pltpu.SemaphoreType.DMA,
            pltpu.SemaphoreType.DMA,
            pltpu.SemaphoreType.DMA,
        ],
    )
    def sc_kernel(rid_hbm, table_hbm, posT_hbm, out_hbm,
                  idx0, idx1, rows0, rows1, posT_v, outT0, outT1,
                  semi0, semi1, semg0, semg1, semo0, semo1):
        idx_v = (idx0, idx1)
        rows_v = (rows0, rows1)
        outT_v = (outT0, outT1)
        semi = (semi0, semi1)
        semg = (semg0, semg1)
        semo = (semo0, semo1)

        wid = lax.axis_index("s") * NUM_CORES + lax.axis_index("c")
        base = wid * b_per_w
        pltpu.sync_copy(posT_hbm, posT_v)
        iota = lax.iota(jnp.int32, LANES)
        dbases = [(iota + d0 * LANES) * T for d0 in range(n_dc)]

        def start_gather(buf):
            for off, n in chunks:
                pltpu.async_copy(
                    table_hbm.at[idx_v[buf].at[pl.ds(off, n)]],
                    rows_v[buf].at[pl.ds(off, n)], semg[buf])

        def drain_gather(buf):
            pltpu.make_async_copy(
                table_hbm.at[pl.ds(0, T)], rows_v[buf], semg[buf]).wait()

        for buf in (0, 1):
            pltpu.sync_copy(rid_hbm.at[base + buf], idx_v[buf])
            start_gather(buf)

        def body(j, carry):
            for buf in (0, 1):
                b = base + 2 * j + buf
                bn = jnp.minimum(b + 2, B - 1)
                drain_gather(buf)
                pltpu.async_copy(rid_hbm.at[bn], idx_v[buf], semi[buf])
                @pl.when(j > 0)
                def _():
                    pltpu.make_async_copy(
                        outT_v[buf], out_hbm.at[b], semo[buf]).wait()

                @plsc.parallel_loop(0, T, step=1, unroll=8)
                def tbody(t):
                    for d0 in range(n_dc):
                        vec = (rows_v[buf][t, pl.ds(d0 * LANES, LANES)]
                               + posT_v[t, pl.ds(d0 * LANES, LANES)])
                        plsc.store_scatter(outT_v[buf], [dbases[d0] + t], vec)
                pltpu.async_copy(outT_v[buf], out_hbm.at[b], semo[buf])
                pltpu.make_async_copy(
                    rid_hbm.at[bn], idx_v[buf], semi[buf]).wait()
                start_gather(buf)
            return carry

        lax.fori_loop(0, b_per_w // 2, body, 0)

        for buf in (0, 1):
            drain_gather(buf)
            pltpu.make_async_copy(
                outT_v[buf], out_hbm.at[base], semo[buf]).wait()

    return sc_kernel(rid, table, posT)


def kernel(rid, table, pos):
    B, T = rid.shape
    N, D = table.shape
    rid = rid.astype(jnp.int32)
    posT = jnp.transpose(pos[0].astype(jnp.float32))
    out = _sc_encode(rid, table, posT, B=B, T=T, N=N, D=D)
    return out.reshape(B, D, T)
